# Initial kernel scaffold; baseline (speedup 1.0000x reference)
import jax
import jax.numpy as jnp
from jax.experimental import pallas as pl


def kernel(pos_u, pos_v, neg_v, u_weight, v_weight):
    raise NotImplementedError("write your pallas kernel here")


# speedup vs baseline: 1.4955x; 1.4955x over previous
import functools

import jax
import jax.numpy as jnp
from jax import lax
from jax.experimental import pallas as pl
from jax.experimental.pallas import tpu as pltpu
from jax.experimental.pallas import tpu_sc as plsc

_VOCAB = 100000
_DIM = 128
_BATCH = 16384
_NEG = 20
_LANES = 16

_NW = 32
_CB = _BATCH // _NW
_CH = 16
_NCH = _CB // _CH
_NROWS = 24


def _sc_body(pos_u_hbm, pos_v_hbm, neg_hbm, u_w_hbm, v_w_hbm, out_hbm,
             idx_u, idx_v, idx_n, u_buf, v_buf, n_buf, score_v, sem):
    nc = 2
    wid = lax.axis_index("s") * nc + lax.axis_index("c")
    base = wid * _CB

    pltpu.sync_copy(pos_u_hbm.at[pl.ds(base, _CB)], idx_u)
    pltpu.sync_copy(pos_v_hbm.at[pl.ds(base, _CB)], idx_v)
    pltpu.sync_copy(neg_hbm.at[pl.ds(base * _NEG, _CB * _NEG)], idx_n)

    lane = lax.iota(jnp.int32, _LANES)
    rows_n = [lane * _NEG + k for k in range(_NEG)]

    def chunk_body(i, carry):
        b0 = i * _CH
        n0 = i * (_CH * _NEG)

        cps = [
            pltpu.async_copy(u_w_hbm.at[idx_u.at[pl.ds(b0, _CH)]],
                             u_buf, sem),
            pltpu.async_copy(v_w_hbm.at[idx_v.at[pl.ds(b0, _CH)]],
                             v_buf, sem),
            pltpu.async_copy(v_w_hbm.at[idx_n.at[pl.ds(n0, 128)]],
                             n_buf.at[pl.ds(0, 128)], sem),
            pltpu.async_copy(v_w_hbm.at[idx_n.at[pl.ds(n0 + 128, 128)]],
                             n_buf.at[pl.ds(128, 128)], sem),
            pltpu.async_copy(v_w_hbm.at[idx_n.at[pl.ds(n0 + 256, 64)]],
                             n_buf.at[pl.ds(256, 64)], sem),
        ]
        for cp in cps:
            cp.wait()

        def d_body(d, accs):
            col = jnp.full((_LANES,), d, jnp.int32)
            u_d = plsc.load_gather(u_buf, [lane, col])
            v_d = plsc.load_gather(v_buf, [lane, col])
            new = [accs[0] + u_d * v_d]
            for k in range(_NEG):
                n_d = plsc.load_gather(n_buf, [rows_n[k], col])
                new.append(accs[k + 1] + u_d * n_d)
            return tuple(new)

        accs0 = tuple(jnp.zeros((_LANES,), jnp.float32) for _ in range(_NEG + 1))
        accs = lax.fori_loop(0, _DIM, d_body, accs0)

        for r in range(_NEG + 1):
            score_v[r, pl.ds(b0, _CH)] = accs[r]
        return carry

    lax.fori_loop(0, _NCH, chunk_body, 0)

    zero = jnp.zeros((_LANES,), jnp.float32)
    for r in range(_NEG + 1, _NROWS):
        def pad_body(i, carry, r=r):
            score_v[r, pl.ds(i * _LANES, _LANES)] = zero
            return carry
        lax.fori_loop(0, _CB // _LANES, pad_body, 0)

    pltpu.sync_copy(score_v, out_hbm.at[:, pl.ds(base, _CB)])


_sc_scores = functools.partial(
    pl.kernel,
    out_type=jax.ShapeDtypeStruct((_NROWS, _BATCH), jnp.float32),
    mesh=plsc.VectorSubcoreMesh(core_axis_name="c", subcore_axis_name="s"),
    scratch_types=[
        pltpu.VMEM((_CB,), jnp.int32),
        pltpu.VMEM((_CB,), jnp.int32),
        pltpu.VMEM((_CB * _NEG,), jnp.int32),
        pltpu.VMEM((_CH, _DIM), jnp.float32),
        pltpu.VMEM((_CH, _DIM), jnp.float32),
        pltpu.VMEM((_CH * _NEG, _DIM), jnp.float32),
        pltpu.VMEM((_NROWS, _CB), jnp.float32),
        pltpu.SemaphoreType.DMA,
    ],
    compiler_params=pltpu.CompilerParams(needs_layout_passes=False),
)(_sc_body)


def _tc_loss_body(s_ref, o_ref):
    x = s_ref[...]
    row = lax.broadcasted_iota(jnp.int32, x.shape, 0)
    y = jax.nn.log_sigmoid(jnp.where(row == 0, x, -x))
    y = jnp.where(row < _NEG + 1, y, 0.0)
    w = jnp.where(row == 0, 1.0 / _BATCH,
                  jnp.where(row < _NEG + 1, 1.0 / (_BATCH * _NEG), 0.0))
    o_ref[0, 0] = -jnp.sum(y * w.astype(jnp.float32))


_tc_loss = pl.pallas_call(
    _tc_loss_body,
    out_shape=jax.ShapeDtypeStruct((1, 1), jnp.float32),
    out_specs=pl.BlockSpec(memory_space=pltpu.SMEM),
)


@jax.jit
def kernel(pos_u, pos_v, neg_v, u_weight, v_weight):
    pos_u = pos_u.astype(jnp.int32)
    pos_v = pos_v.astype(jnp.int32)
    neg_flat = neg_v.astype(jnp.int32).reshape(-1)
    scores = _sc_scores(pos_u, pos_v, neg_flat, u_weight, v_weight)
    return _tc_loss(scores)[0, 0]


# speedup vs baseline: 1.5007x; 1.0035x over previous
import functools

import jax
import jax.numpy as jnp
from jax import lax
from jax.experimental import pallas as pl
from jax.experimental.pallas import tpu as pltpu
from jax.experimental.pallas import tpu_sc as plsc

_VOCAB = 100000
_DIM = 128
_BATCH = 16384
_NEG = 20
_LANES = 16

_NW = 32
_CB = _BATCH // _NW
_CH = 16
_NCH = _CB // _CH
_VN = _CH * (1 + _NEG)
_NROWS = 24


def _sc_body(pos_u_hbm, vn_idx_hbm, u_w_hbm, v_w_hbm, out_hbm,
             idx_u, idx_vn, u_buf, vn_buf, score_v, sem0, sem1):
    nc = 2
    wid = lax.axis_index("s") * nc + lax.axis_index("c")
    base = wid * _CB

    pltpu.sync_copy(pos_u_hbm.at[pl.ds(base, _CB)], idx_u)
    pltpu.sync_copy(vn_idx_hbm.at[pl.ds(wid * (_NCH * _VN), _NCH * _VN)],
                    idx_vn)

    sems = (sem0, sem1)

    def dmas(c, slot):
        o = c * _VN
        return [
            (u_w_hbm.at[idx_u.at[pl.ds(c * _CH, _CH)]],
             u_buf.at[slot], sems[slot]),
            (v_w_hbm.at[idx_vn.at[pl.ds(o, 128)]],
             vn_buf.at[slot, pl.ds(0, 128)], sems[slot]),
            (v_w_hbm.at[idx_vn.at[pl.ds(o + 128, 128)]],
             vn_buf.at[slot, pl.ds(128, 128)], sems[slot]),
            (v_w_hbm.at[idx_vn.at[pl.ds(o + 256, _VN - 256)]],
             vn_buf.at[slot, pl.ds(256, _VN - 256)], sems[slot]),
        ]

    def fire(c, slot):
        for s, d, m in dmas(c, slot):
            pltpu.async_copy(s, d, m)

    def drain(c, slot):
        for s, d, m in dmas(c, slot):
            pltpu.make_async_copy(s, d, m).wait()

    lane = lax.iota(jnp.int32, _LANES)
    nrow_base = lane * _NEG + _CH

    def compute(c, slot):
        ub = u_buf.at[slot]
        vb = vn_buf.at[slot]

        def d_body(j, accs):
            accs = list(accs)
            for jj in range(8):
                d = j * 8 + jj
                col = jnp.full((_LANES,), d, jnp.int32)
                u_d = plsc.load_gather(ub, [lane, col])
                v_d = plsc.load_gather(vb, [lane, col])
                accs[0] = accs[0] + u_d * v_d
                for k in range(_NEG):
                    n_d = plsc.load_gather(vb, [nrow_base + k, col])
                    accs[k + 1] = accs[k + 1] + u_d * n_d
            return tuple(accs)

        accs0 = tuple(jnp.zeros((_LANES,), jnp.float32)
                      for _ in range(_NEG + 1))
        accs = lax.fori_loop(0, _DIM // 8, d_body, accs0)
        b0 = c * _CH
        for r in range(_NEG + 1):
            score_v[r, pl.ds(b0, _CH)] = accs[r]

    fire(0, 0)

    def outer(i, carry):
        c0 = 2 * i
        fire(c0 + 1, 1)
        drain(c0, 0)
        compute(c0, 0)

        @pl.when(i < _NCH // 2 - 1)
        def _():
            fire(c0 + 2, 0)

        drain(c0 + 1, 1)
        compute(c0 + 1, 1)
        return carry

    lax.fori_loop(0, _NCH // 2, outer, 0)

    zero = jnp.zeros((_LANES,), jnp.float32)
    for r in range(_NEG + 1, _NROWS):
        def pad_body(i, carry, r=r):
            score_v[r, pl.ds(i * _LANES, _LANES)] = zero
            return carry
        lax.fori_loop(0, _CB // _LANES, pad_body, 0)

    pltpu.sync_copy(score_v, out_hbm.at[:, pl.ds(base, _CB)])


_sc_scores = functools.partial(
    pl.kernel,
    out_type=jax.ShapeDtypeStruct((_NROWS, _BATCH), jnp.float32),
    mesh=plsc.VectorSubcoreMesh(core_axis_name="c", subcore_axis_name="s"),
    scratch_types=[
        pltpu.VMEM((_CB,), jnp.int32),
        pltpu.VMEM((_NCH * _VN,), jnp.int32),
        pltpu.VMEM((2, _CH, _DIM), jnp.float32),
        pltpu.VMEM((2, _VN, _DIM), jnp.float32),
        pltpu.VMEM((_NROWS, _CB), jnp.float32),
        pltpu.SemaphoreType.DMA,
        pltpu.SemaphoreType.DMA,
    ],
    compiler_params=pltpu.CompilerParams(needs_layout_passes=False),
)(_sc_body)


def _tc_loss_body(s_ref, o_ref):
    x = s_ref[...]
    row = lax.broadcasted_iota(jnp.int32, x.shape, 0)
    y = jax.nn.log_sigmoid(jnp.where(row == 0, x, -x))
    y = jnp.where(row < _NEG + 1, y, 0.0)
    w = jnp.where(row == 0, 1.0 / _BATCH,
                  jnp.where(row < _NEG + 1, 1.0 / (_BATCH * _NEG), 0.0))
    o_ref[0, 0] = -jnp.sum(y * w.astype(jnp.float32))


_tc_loss = pl.pallas_call(
    _tc_loss_body,
    out_shape=jax.ShapeDtypeStruct((1, 1), jnp.float32),
    out_specs=pl.BlockSpec(memory_space=pltpu.SMEM),
)


@jax.jit
def kernel(pos_u, pos_v, neg_v, u_weight, v_weight):
    pos_u = pos_u.astype(jnp.int32)
    pos_v = pos_v.astype(jnp.int32)
    neg_flat = neg_v.astype(jnp.int32).reshape(_BATCH // _CH, _CH * _NEG)
    vn_idx = jnp.concatenate(
        [pos_v.reshape(_BATCH // _CH, _CH), neg_flat], axis=1).reshape(-1)
    scores = _sc_scores(pos_u, vn_idx, u_weight, v_weight)
    return _tc_loss(scores)[0, 0]


# speedup vs baseline: 4.0910x; 2.7260x over previous
import functools

import jax
import jax.numpy as jnp
from jax import lax
from jax.experimental import pallas as pl
from jax.experimental.pallas import tpu as pltpu
from jax.experimental.pallas import tpu_sc as plsc

_VOCAB = 100000
_DIM = 128
_BATCH = 16384
_NEG = 20
_LANES = 16

_NW = 32
_CB = _BATCH // _NW
_CH = 16
_NCH = _CB // _CH
_VN = _CH * (1 + _NEG)
_NROWS = 24


def _sc_body(pos_u_hbm, vn_idx_hbm, u_w_hbm, v_w_hbm, out_hbm,
             idx_u, idx_vn, u_buf, vn_buf, score_v, sem0, sem1):
    nc = 2
    wid = lax.axis_index("s") * nc + lax.axis_index("c")
    base = wid * _CB

    pltpu.sync_copy(pos_u_hbm.at[pl.ds(base, _CB)], idx_u)
    pltpu.sync_copy(vn_idx_hbm.at[pl.ds(wid * (_NCH * _VN), _NCH * _VN)],
                    idx_vn)

    sems = (sem0, sem1)

    def dmas(c, slot):
        o = c * _VN
        return [
            (u_w_hbm.at[idx_u.at[pl.ds(c * _CH, _CH)]],
             u_buf.at[slot], sems[slot]),
            (v_w_hbm.at[idx_vn.at[pl.ds(o, 128)]],
             vn_buf.at[slot, pl.ds(0, 128)], sems[slot]),
            (v_w_hbm.at[idx_vn.at[pl.ds(o + 128, 128)]],
             vn_buf.at[slot, pl.ds(128, 128)], sems[slot]),
            (v_w_hbm.at[idx_vn.at[pl.ds(o + 256, _VN - 256)]],
             vn_buf.at[slot, pl.ds(256, _VN - 256)], sems[slot]),
        ]

    def fire(c, slot):
        for s, d, m in dmas(c, slot):
            pltpu.async_copy(s, d, m)

    def drain(c, slot):
        for s, d, m in dmas(c, slot):
            pltpu.make_async_copy(s, d, m).wait()

    lane = lax.iota(jnp.int32, _LANES)
    nrow_base = lane * _NEG + _CH

    def compute(c, slot):
        ub = u_buf.at[slot]
        vb = vn_buf.at[slot]

        def d_body(j, accs):
            accs = list(accs)
            for jj in range(8):
                d = j * 8 + jj
                col = jnp.bitwise_and(lane + d, _DIM - 1)
                u_d = plsc.load_gather(ub, [lane, col])
                v_d = plsc.load_gather(vb, [lane, col])
                accs[0] = accs[0] + u_d * v_d
                for k in range(_NEG):
                    n_d = plsc.load_gather(vb, [nrow_base + k, col])
                    accs[k + 1] = accs[k + 1] + u_d * n_d
            return tuple(accs)

        accs0 = tuple(jnp.zeros((_LANES,), jnp.float32)
                      for _ in range(_NEG + 1))
        accs = lax.fori_loop(0, _DIM // 8, d_body, accs0)
        b0 = c * _CH
        for r in range(_NEG + 1):
            score_v[r, pl.ds(b0, _CH)] = accs[r]

    fire(0, 0)

    def outer(i, carry):
        c0 = 2 * i
        fire(c0 + 1, 1)
        drain(c0, 0)
        compute(c0, 0)

        @pl.when(i < _NCH // 2 - 1)
        def _():
            fire(c0 + 2, 0)

        drain(c0 + 1, 1)
        compute(c0 + 1, 1)
        return carry

    lax.fori_loop(0, _NCH // 2, outer, 0)

    zero = jnp.zeros((_LANES,), jnp.float32)
    for r in range(_NEG + 1, _NROWS):
        def pad_body(i, carry, r=r):
            score_v[r, pl.ds(i * _LANES, _LANES)] = zero
            return carry
        lax.fori_loop(0, _CB // _LANES, pad_body, 0)

    pltpu.sync_copy(score_v, out_hbm.at[:, pl.ds(base, _CB)])


_sc_scores = functools.partial(
    pl.kernel,
    out_type=jax.ShapeDtypeStruct((_NROWS, _BATCH), jnp.float32),
    mesh=plsc.VectorSubcoreMesh(core_axis_name="c", subcore_axis_name="s"),
    scratch_types=[
        pltpu.VMEM((_CB,), jnp.int32),
        pltpu.VMEM((_NCH * _VN,), jnp.int32),
        pltpu.VMEM((2, _CH, _DIM), jnp.float32),
        pltpu.VMEM((2, _VN, _DIM), jnp.float32),
        pltpu.VMEM((_NROWS, _CB), jnp.float32),
        pltpu.SemaphoreType.DMA,
        pltpu.SemaphoreType.DMA,
    ],
    compiler_params=pltpu.CompilerParams(needs_layout_passes=False),
)(_sc_body)


def _tc_loss_body(s_ref, o_ref):
    x = s_ref[...]
    row = lax.broadcasted_iota(jnp.int32, x.shape, 0)
    y = jax.nn.log_sigmoid(jnp.where(row == 0, x, -x))
    y = jnp.where(row < _NEG + 1, y, 0.0)
    w = jnp.where(row == 0, 1.0 / _BATCH,
                  jnp.where(row < _NEG + 1, 1.0 / (_BATCH * _NEG), 0.0))
    o_ref[0, 0] = -jnp.sum(y * w.astype(jnp.float32))


_tc_loss = pl.pallas_call(
    _tc_loss_body,
    out_shape=jax.ShapeDtypeStruct((1, 1), jnp.float32),
    out_specs=pl.BlockSpec(memory_space=pltpu.SMEM),
)


@jax.jit
def kernel(pos_u, pos_v, neg_v, u_weight, v_weight):
    pos_u = pos_u.astype(jnp.int32)
    pos_v = pos_v.astype(jnp.int32)
    neg_flat = neg_v.astype(jnp.int32).reshape(_BATCH // _CH, _CH * _NEG)
    vn_idx = jnp.concatenate(
        [pos_v.reshape(_BATCH // _CH, _CH), neg_flat], axis=1).reshape(-1)
    scores = _sc_scores(pos_u, vn_idx, u_weight, v_weight)
    return _tc_loss(scores)[0, 0]


# speedup vs baseline: 11.8381x; 2.8937x over previous
import functools

import jax
import jax.numpy as jnp
from jax import lax
from jax.experimental import pallas as pl
from jax.experimental.pallas import tpu as pltpu
from jax.experimental.pallas import tpu_sc as plsc

_VOCAB = 100000
_DIM = 128
_BATCH = 16384
_NEG = 20
_LANES = 16

_NW = 32
_CB = _BATCH // _NW
_CH = 16
_NCH = _CB // _CH
_VN = _CH * (1 + _NEG)
_NROWS = 24


def _sc_body(pos_u_hbm, vn_idx_hbm, u_w_hbm, v_w_hbm, out_hbm,
             idx_u, idx_vn, u_buf, vn_buf, score_v, sem0, sem1):
    nc = 2
    wid = lax.axis_index("s") * nc + lax.axis_index("c")
    base = wid * _CB

    pltpu.sync_copy(pos_u_hbm.at[pl.ds(base, _CB)], idx_u)
    pltpu.sync_copy(vn_idx_hbm.at[pl.ds(wid * (_NCH * _VN), _NCH * _VN)],
                    idx_vn)

    sems = (sem0, sem1)

    def dmas(c, slot):
        o = c * _VN
        return [
            (u_w_hbm.at[idx_u.at[pl.ds(c * _CH, _CH)]],
             u_buf.at[slot], sems[slot]),
            (v_w_hbm.at[idx_vn.at[pl.ds(o, 128)]],
             vn_buf.at[slot, pl.ds(0, 128)], sems[slot]),
            (v_w_hbm.at[idx_vn.at[pl.ds(o + 128, 128)]],
             vn_buf.at[slot, pl.ds(128, 128)], sems[slot]),
            (v_w_hbm.at[idx_vn.at[pl.ds(o + 256, _VN - 256)]],
             vn_buf.at[slot, pl.ds(256, _VN - 256)], sems[slot]),
        ]

    def fire(c, slot):
        for s, d, m in dmas(c, slot):
            pltpu.async_copy(s, d, m)

    def drain(c, slot):
        for s, d, m in dmas(c, slot):
            pltpu.make_async_copy(s, d, m).wait()

    lane = lax.iota(jnp.int32, _LANES)
    last_lane = lane == (_LANES - 1)
    nvec = _DIM // _LANES

    def compute(c, slot):
        ub = u_buf.at[slot]
        vb = vn_buf.at[slot]

        def b_body(b, carry):
            pos = jnp.full((_LANES,), c * _CH, jnp.int32) + b
            nrow = b * _NEG + _CH
            rows = [b] + [nrow + k for k in range(_NEG)]
            accs = [None] * len(rows)
            for j in range(nvec):
                sl = pl.ds(j * _LANES, _LANES)
                u_j = ub[b, sl]
                for r, row in enumerate(rows):
                    t = u_j * vb[row, sl]
                    accs[r] = t if j == 0 else accs[r] + t
            for r in range(len(rows)):
                plsc.store_scatter(score_v, [pos + r * _CB],
                                   plsc.cumsum(accs[r]), mask=last_lane)
            return carry

        lax.fori_loop(0, _CH, b_body, 0)

    fire(0, 0)

    def outer(i, carry):
        c0 = 2 * i
        fire(c0 + 1, 1)
        drain(c0, 0)
        compute(c0, 0)

        @pl.when(i < _NCH // 2 - 1)
        def _():
            fire(c0 + 2, 0)

        drain(c0 + 1, 1)
        compute(c0 + 1, 1)
        return carry

    lax.fori_loop(0, _NCH // 2, outer, 0)

    zero = jnp.zeros((_LANES,), jnp.float32)
    for r in range(_NEG + 1, _NROWS):
        def pad_body(i, carry, r=r):
            score_v[pl.ds(r * _CB + i * _LANES, _LANES)] = zero
            return carry
        lax.fori_loop(0, _CB // _LANES, pad_body, 0)

    for r in range(_NROWS):
        pltpu.sync_copy(score_v.at[pl.ds(r * _CB, _CB)],
                        out_hbm.at[r, pl.ds(base, _CB)])


_sc_scores = functools.partial(
    pl.kernel,
    out_type=jax.ShapeDtypeStruct((_NROWS, _BATCH), jnp.float32),
    mesh=plsc.VectorSubcoreMesh(core_axis_name="c", subcore_axis_name="s"),
    scratch_types=[
        pltpu.VMEM((_CB,), jnp.int32),
        pltpu.VMEM((_NCH * _VN,), jnp.int32),
        pltpu.VMEM((2, _CH, _DIM), jnp.float32),
        pltpu.VMEM((2, _VN, _DIM), jnp.float32),
        pltpu.VMEM((_NROWS * _CB,), jnp.float32),
        pltpu.SemaphoreType.DMA,
        pltpu.SemaphoreType.DMA,
    ],
    compiler_params=pltpu.CompilerParams(needs_layout_passes=False),
)(_sc_body)


def _tc_loss_body(s_ref, o_ref):
    x = s_ref[...]
    row = lax.broadcasted_iota(jnp.int32, x.shape, 0)
    y = jax.nn.log_sigmoid(jnp.where(row == 0, x, -x))
    y = jnp.where(row < _NEG + 1, y, 0.0)
    w = jnp.where(row == 0, 1.0 / _BATCH,
                  jnp.where(row < _NEG + 1, 1.0 / (_BATCH * _NEG), 0.0))
    o_ref[0, 0] = -jnp.sum(y * w.astype(jnp.float32))


_tc_loss = pl.pallas_call(
    _tc_loss_body,
    out_shape=jax.ShapeDtypeStruct((1, 1), jnp.float32),
    out_specs=pl.BlockSpec(memory_space=pltpu.SMEM),
)


@jax.jit
def kernel(pos_u, pos_v, neg_v, u_weight, v_weight):
    pos_u = pos_u.astype(jnp.int32)
    pos_v = pos_v.astype(jnp.int32)
    neg_flat = neg_v.astype(jnp.int32).reshape(_BATCH // _CH, _CH * _NEG)
    vn_idx = jnp.concatenate(
        [pos_v.reshape(_BATCH // _CH, _CH), neg_flat], axis=1).reshape(-1)
    scores = _sc_scores(pos_u, vn_idx, u_weight, v_weight)
    return _tc_loss(scores)[0, 0]


# speedup vs baseline: 12.0642x; 1.0191x over previous
import functools

import jax
import jax.numpy as jnp
from jax import lax
from jax.experimental import pallas as pl
from jax.experimental.pallas import tpu as pltpu
from jax.experimental.pallas import tpu_sc as plsc

_VOCAB = 100000
_DIM = 128
_BATCH = 16384
_NEG = 20
_LANES = 16

_NW = 32
_CB = _BATCH // _NW
_CH = 16
_NCH = _CB // _CH
_VN = _CH * (1 + _NEG)
_NROWS = 24


def _sc_body(pos_u_hbm, vn_idx_hbm, u_w_hbm, v_w_hbm, out_hbm,
             idx_u, idx_vn, u_buf, vn_buf, score_v, sem0, sem1):
    nc = 2
    wid = lax.axis_index("s") * nc + lax.axis_index("c")
    base = wid * _CB

    st1 = pltpu.async_copy(pos_u_hbm.at[pl.ds(base, _CB)], idx_u, sem0)
    st2 = pltpu.async_copy(
        vn_idx_hbm.at[pl.ds(wid * (_NCH * _VN), _NCH * _VN)], idx_vn, sem1)
    st1.wait()
    st2.wait()

    sems = (sem0, sem1)

    def dmas(c, slot):
        return [
            (u_w_hbm.at[idx_u.at[pl.ds(c * _CH, _CH)]],
             u_buf.at[slot], sems[slot]),
            (v_w_hbm.at[idx_vn.at[pl.ds(c * _VN, _VN)]],
             vn_buf.at[slot], sems[slot]),
        ]

    def fire(c, slot):
        for s, d, m in dmas(c, slot):
            pltpu.async_copy(s, d, m)

    def drain(c, slot):
        for s, d, m in dmas(c, slot):
            pltpu.make_async_copy(s, d, m).wait()

    lane = lax.iota(jnp.int32, _LANES)
    last_lane = lane == (_LANES - 1)
    nvec = _DIM // _LANES

    def compute(c, slot):
        ub = u_buf.at[slot]
        vb = vn_buf.at[slot]

        def b_body(b, carry):
            pos = jnp.full((_LANES,), c * _CH, jnp.int32) + b
            nrow = b * _NEG + _CH
            rows = [b] + [nrow + k for k in range(_NEG)]
            accs = [None] * len(rows)
            for j in range(nvec):
                sl = pl.ds(j * _LANES, _LANES)
                u_j = ub[b, sl]
                for r, row in enumerate(rows):
                    t = u_j * vb[row, sl]
                    accs[r] = t if j == 0 else accs[r] + t
            for r in range(len(rows)):
                plsc.store_scatter(score_v, [pos + r * _CB],
                                   plsc.cumsum(accs[r]), mask=last_lane)
            return carry

        lax.fori_loop(0, _CH, b_body, 0)

    fire(0, 0)

    def outer(i, carry):
        c0 = 2 * i
        fire(c0 + 1, 1)
        drain(c0, 0)
        compute(c0, 0)

        @pl.when(i < _NCH // 2 - 1)
        def _():
            fire(c0 + 2, 0)

        drain(c0 + 1, 1)
        compute(c0 + 1, 1)
        return carry

    lax.fori_loop(0, _NCH // 2, outer, 0)

    zero = jnp.zeros((_LANES,), jnp.float32)
    for r in range(_NEG + 1, _NROWS):
        def pad_body(i, carry, r=r):
            score_v[pl.ds(r * _CB + i * _LANES, _LANES)] = zero
            return carry
        lax.fori_loop(0, _CB // _LANES, pad_body, 0)

    pltpu.sync_copy(score_v, out_hbm.at[wid])


_sc_scores = functools.partial(
    pl.kernel,
    out_type=jax.ShapeDtypeStruct((_NW, _NROWS * _CB), jnp.float32),
    mesh=plsc.VectorSubcoreMesh(core_axis_name="c", subcore_axis_name="s"),
    scratch_types=[
        pltpu.VMEM((_CB,), jnp.int32),
        pltpu.VMEM((_NCH * _VN,), jnp.int32),
        pltpu.VMEM((2, _CH, _DIM), jnp.float32),
        pltpu.VMEM((2, _VN, _DIM), jnp.float32),
        pltpu.VMEM((_NROWS * _CB,), jnp.float32),
        pltpu.SemaphoreType.DMA,
        pltpu.SemaphoreType.DMA,
    ],
    compiler_params=pltpu.CompilerParams(needs_layout_passes=False),
)(_sc_body)


def _tc_loss_body(s_ref, o_ref):
    x = s_ref[...]
    col = lax.broadcasted_iota(jnp.int32, x.shape, 1)
    row = col // _CB
    y = jax.nn.log_sigmoid(jnp.where(row == 0, x, -x))
    y = jnp.where(row < _NEG + 1, y, 0.0)
    w = jnp.where(row == 0, 1.0 / _BATCH,
                  jnp.where(row < _NEG + 1, 1.0 / (_BATCH * _NEG), 0.0))
    o_ref[0, 0] = -jnp.sum(y * w.astype(jnp.float32))


_tc_loss = pl.pallas_call(
    _tc_loss_body,
    out_shape=jax.ShapeDtypeStruct((1, 1), jnp.float32),
    out_specs=pl.BlockSpec(memory_space=pltpu.SMEM),
)


@jax.jit
def kernel(pos_u, pos_v, neg_v, u_weight, v_weight):
    pos_u = pos_u.astype(jnp.int32)
    pos_v = pos_v.astype(jnp.int32)
    neg_flat = neg_v.astype(jnp.int32).reshape(_BATCH // _CH, _CH * _NEG)
    vn_idx = jnp.concatenate(
        [pos_v.reshape(_BATCH // _CH, _CH), neg_flat], axis=1).reshape(-1)
    scores = _sc_scores(pos_u, vn_idx, u_weight, v_weight)
    return _tc_loss(scores)[0, 0]
